# baseline (speedup 1.0000x reference)
import functools

import jax
import jax.numpy as jnp
from jax.experimental import pallas as pl
from jax.experimental.pallas import tpu as pltpu

N_TOKENS = 8192
D = 2048
H = 512
E = 8
BN = 512

_INV_SQRT2 = 0.7071067811865476


def _moe_head_kernel(x_ref, gw_ref, gb_ref, w1c_ref, b1c_ref, w2b_ref,
                     scores_ref, logits_ref):
    xb = x_ref[...].astype(jnp.bfloat16)
    logits = jax.lax.dot_general(
        xb, gw_ref[...], (((1,), (1,)), ((), ())),
        preferred_element_type=jnp.float32,
    ) + gb_ref[...]
    logits_ref[...] = logits

    cols = jax.lax.broadcasted_iota(jnp.int32, (BN, E), 1)
    m1 = jnp.max(logits, axis=1, keepdims=True)
    i1 = jnp.min(jnp.where(logits == m1, cols, E), axis=1, keepdims=True)
    masked = jnp.where(cols == i1, -jnp.inf, logits)
    m2 = jnp.max(masked, axis=1, keepdims=True)
    i2 = jnp.min(jnp.where(masked == m2, cols, E), axis=1, keepdims=True)
    s = jax.nn.sigmoid(m1 - m2)

    NC = E * H // 4
    pscores = jnp.zeros((BN, E), jnp.float32)
    for j in range(0, E * H, NC):
        aj = jax.lax.dot_general(
            xb, w1c_ref[j:j + NC, :], (((1,), (1,)), ((), ())),
            preferred_element_type=jnp.float32,
        ) + b1c_ref[:, j:j + NC]
        gj = 0.5 * aj * (1.0 + jax.lax.erf(aj * _INV_SQRT2))
        pscores = pscores + jax.lax.dot_general(
            gj, w2b_ref[j:j + NC, :], (((1,), (0,)), ((), ())),
            preferred_element_type=jnp.float32,
        )
    wexp = (jnp.where(cols == i1, s, 0.0)
            + jnp.where(cols == i2, 1.0 - s, 0.0))
    scores_ref[...] = jnp.sum(pscores * wexp, axis=1, keepdims=True)


@functools.partial(jax.jit, static_argnames=())
def kernel(x, gate_w, gate_b, w1, b1, w2):
    w1c = w1.reshape(E * H, D).astype(jnp.bfloat16)
    b1c = b1.reshape(1, E * H)
    idx = jnp.arange(E)
    w2b = (jnp.zeros((E, H, E), jnp.float32)
           .at[idx, :, idx].set(w2.reshape(E, H))
           .reshape(E * H, E))
    gwb = gate_w.astype(jnp.bfloat16)
    gb2 = gate_b.reshape(1, E)
    grid = (N_TOKENS // BN,)
    scores, logits = pl.pallas_call(
        _moe_head_kernel,
        grid=grid,
        in_specs=[
            pl.BlockSpec((BN, D), lambda i: (i, 0)),
            pl.BlockSpec((E, D), lambda i: (0, 0)),
            pl.BlockSpec((1, E), lambda i: (0, 0)),
            pl.BlockSpec((E * H, D), lambda i: (0, 0)),
            pl.BlockSpec((1, E * H), lambda i: (0, 0)),
            pl.BlockSpec((E * H, E), lambda i: (0, 0)),
        ],
        out_specs=[
            pl.BlockSpec((BN, 1), lambda i: (i, 0)),
            pl.BlockSpec((BN, E), lambda i: (i, 0)),
        ],
        out_shape=[
            jax.ShapeDtypeStruct((N_TOKENS, 1), jnp.float32),
            jax.ShapeDtypeStruct((N_TOKENS, E), jnp.float32),
        ],
        compiler_params=pltpu.CompilerParams(
            dimension_semantics=("parallel",),
        ),
    )(x, gwb, gb2, w1c, b1c, w2b)
    return scores, logits


# speedup vs baseline: 1.1423x; 1.1423x over previous
import functools

import jax
import jax.numpy as jnp
from jax.experimental import pallas as pl
from jax.experimental.pallas import tpu as pltpu

N_TOKENS = 8192
D = 2048
H = 512
E = 8
BN = 512

_INV_SQRT2 = 0.7071067811865476


_WCHUNK = 512


def _moe_head_kernel(x_ref, gw_ref, gb_ref, w1c_ref, b1c_ref, w2b_ref,
                     scores_ref, logits_ref, w1s_ref, stg_ref, dma_sem):
    @pl.when(pl.program_id(0) == 0)
    def _cast_weights():
        def body(k, carry):
            cp = pltpu.make_async_copy(
                w1c_ref.at[pl.ds(k * _WCHUNK, _WCHUNK), :], stg_ref, dma_sem)
            cp.start()
            cp.wait()
            w1s_ref[pl.ds(k * _WCHUNK, _WCHUNK), :] = (
                stg_ref[...].astype(jnp.bfloat16))
            return carry
        jax.lax.fori_loop(0, E * H // _WCHUNK, body, 0)

    xb = x_ref[...].astype(jnp.bfloat16)
    logits = jax.lax.dot_general(
        xb, gw_ref[...], (((1,), (1,)), ((), ())),
        preferred_element_type=jnp.float32,
    ) + gb_ref[...]
    logits_ref[...] = logits

    cols = jax.lax.broadcasted_iota(jnp.int32, (BN, E), 1)
    m1 = jnp.max(logits, axis=1, keepdims=True)
    i1 = jnp.min(jnp.where(logits == m1, cols, E), axis=1, keepdims=True)
    masked = jnp.where(cols == i1, -jnp.inf, logits)
    m2 = jnp.max(masked, axis=1, keepdims=True)
    i2 = jnp.min(jnp.where(masked == m2, cols, E), axis=1, keepdims=True)
    s = jax.nn.sigmoid(m1 - m2)

    a = jax.lax.dot_general(
        xb, w1s_ref[...], (((1,), (1,)), ((), ())),
        preferred_element_type=jnp.float32,
    ) + b1c_ref[...]
    g = (0.5 * a * (1.0 + jax.lax.erf(a * _INV_SQRT2))
         ).astype(jnp.bfloat16)
    pscores = jax.lax.dot_general(
        g, w2b_ref[...], (((1,), (0,)), ((), ())),
        preferred_element_type=jnp.float32,
    )
    wexp = (jnp.where(cols == i1, s, 0.0)
            + jnp.where(cols == i2, 1.0 - s, 0.0))
    scores_ref[...] = jnp.sum(pscores * wexp, axis=1, keepdims=True)


@functools.partial(jax.jit, static_argnames=())
def kernel(x, gate_w, gate_b, w1, b1, w2):
    w1c = w1.reshape(E * H, D)
    b1c = b1.reshape(1, E * H)
    idx = jnp.arange(E)
    w2b = (jnp.zeros((E, H, E), jnp.float32)
           .at[idx, :, idx].set(w2.reshape(E, H))
           .reshape(E * H, E).astype(jnp.bfloat16))
    gwb = gate_w.astype(jnp.bfloat16)
    gb2 = gate_b.reshape(1, E)
    grid = (N_TOKENS // BN,)
    scores, logits = pl.pallas_call(
        _moe_head_kernel,
        grid=grid,
        in_specs=[
            pl.BlockSpec((BN, D), lambda i: (i, 0)),
            pl.BlockSpec((E, D), lambda i: (0, 0)),
            pl.BlockSpec((1, E), lambda i: (0, 0)),
            pl.BlockSpec(memory_space=pl.ANY),
            pl.BlockSpec((1, E * H), lambda i: (0, 0)),
            pl.BlockSpec((E * H, E), lambda i: (0, 0)),
        ],
        out_specs=[
            pl.BlockSpec((BN, 1), lambda i: (i, 0)),
            pl.BlockSpec((BN, E), lambda i: (i, 0)),
        ],
        out_shape=[
            jax.ShapeDtypeStruct((N_TOKENS, 1), jnp.float32),
            jax.ShapeDtypeStruct((N_TOKENS, E), jnp.float32),
        ],
        scratch_shapes=[
            pltpu.VMEM((E * H, D), jnp.bfloat16),
            pltpu.VMEM((_WCHUNK, D), jnp.float32),
            pltpu.SemaphoreType.DMA,
        ],
        compiler_params=pltpu.CompilerParams(
            dimension_semantics=("arbitrary",),
            vmem_limit_bytes=120 * 1024 * 1024,
        ),
    )(x, gwb, gb2, w1c, b1c, w2b)
    return scores, logits


# speedup vs baseline: 1.1503x; 1.0070x over previous
import functools

import jax
import jax.numpy as jnp
from jax.experimental import pallas as pl
from jax.experimental.pallas import tpu as pltpu

N_TOKENS = 8192
D = 2048
H = 512
E = 8
BN = 1024

_INV_SQRT2 = 0.7071067811865476


_WCHUNK = 512


def _moe_head_kernel(x_ref, gw_ref, gb_ref, w1c_ref, b1c_ref, w2b_ref,
                     scores_ref, logits_ref, w1s_ref, stg_ref, dma_sem):
    @pl.when(pl.program_id(0) == 0)
    def _cast_weights():
        def body(k, carry):
            cp = pltpu.make_async_copy(
                w1c_ref.at[pl.ds(k * _WCHUNK, _WCHUNK), :], stg_ref, dma_sem)
            cp.start()
            cp.wait()
            w1s_ref[pl.ds(k * _WCHUNK, _WCHUNK), :] = (
                stg_ref[...].astype(jnp.bfloat16))
            return carry
        jax.lax.fori_loop(0, E * H // _WCHUNK, body, 0)

    xb = x_ref[...].astype(jnp.bfloat16)
    logits = jax.lax.dot_general(
        xb, gw_ref[...], (((1,), (1,)), ((), ())),
        preferred_element_type=jnp.float32,
    ) + gb_ref[...]
    logits_ref[...] = logits

    cols = jax.lax.broadcasted_iota(jnp.int32, (BN, E), 1)
    m1 = jnp.max(logits, axis=1, keepdims=True)
    i1 = jnp.min(jnp.where(logits == m1, cols, E), axis=1, keepdims=True)
    masked = jnp.where(cols == i1, -jnp.inf, logits)
    m2 = jnp.max(masked, axis=1, keepdims=True)
    i2 = jnp.min(jnp.where(masked == m2, cols, E), axis=1, keepdims=True)
    s = jax.nn.sigmoid(m1 - m2)

    a = jax.lax.dot_general(
        xb, w1s_ref[...], (((1,), (1,)), ((), ())),
        preferred_element_type=jnp.float32,
    ) + b1c_ref[...]
    g = (0.5 * a * (1.0 + jax.lax.erf(a * _INV_SQRT2))
         ).astype(jnp.bfloat16)
    pscores = jax.lax.dot_general(
        g, w2b_ref[...], (((1,), (0,)), ((), ())),
        preferred_element_type=jnp.float32,
    )
    wexp = (jnp.where(cols == i1, s, 0.0)
            + jnp.where(cols == i2, 1.0 - s, 0.0))
    scores_ref[...] = jnp.sum(pscores * wexp, axis=1, keepdims=True)


@functools.partial(jax.jit, static_argnames=())
def kernel(x, gate_w, gate_b, w1, b1, w2):
    w1c = w1.reshape(E * H, D)
    b1c = b1.reshape(1, E * H)
    idx = jnp.arange(E)
    w2b = (jnp.zeros((E, H, E), jnp.float32)
           .at[idx, :, idx].set(w2.reshape(E, H))
           .reshape(E * H, E).astype(jnp.bfloat16))
    gwb = gate_w.astype(jnp.bfloat16)
    gb2 = gate_b.reshape(1, E)
    grid = (N_TOKENS // BN,)
    scores, logits = pl.pallas_call(
        _moe_head_kernel,
        grid=grid,
        in_specs=[
            pl.BlockSpec((BN, D), lambda i: (i, 0)),
            pl.BlockSpec((E, D), lambda i: (0, 0)),
            pl.BlockSpec((1, E), lambda i: (0, 0)),
            pl.BlockSpec(memory_space=pl.ANY),
            pl.BlockSpec((1, E * H), lambda i: (0, 0)),
            pl.BlockSpec((E * H, E), lambda i: (0, 0)),
        ],
        out_specs=[
            pl.BlockSpec((BN, 1), lambda i: (i, 0)),
            pl.BlockSpec((BN, E), lambda i: (i, 0)),
        ],
        out_shape=[
            jax.ShapeDtypeStruct((N_TOKENS, 1), jnp.float32),
            jax.ShapeDtypeStruct((N_TOKENS, E), jnp.float32),
        ],
        scratch_shapes=[
            pltpu.VMEM((E * H, D), jnp.bfloat16),
            pltpu.VMEM((_WCHUNK, D), jnp.float32),
            pltpu.SemaphoreType.DMA,
        ],
        compiler_params=pltpu.CompilerParams(
            dimension_semantics=("arbitrary",),
            vmem_limit_bytes=120 * 1024 * 1024,
        ),
    )(x, gwb, gb2, w1c, b1c, w2b)
    return scores, logits


# speedup vs baseline: 1.1981x; 1.0415x over previous
import functools

import jax
import jax.numpy as jnp
from jax.experimental import pallas as pl
from jax.experimental.pallas import tpu as pltpu

N_TOKENS = 8192
D = 2048
H = 512
E = 8
BN = 1024

_INV_SQRT2 = 0.7071067811865476


_WCHUNK = 512


def _moe_head_kernel(x_ref, gw_ref, gb_ref, w1c_ref, b1c_ref, w2b_ref,
                     scores_ref, logits_ref, w1s_ref, stg0_ref, stg1_ref,
                     sem0, sem1):
    @pl.when(pl.program_id(0) == 0)
    def _cast_weights():
        nk = E * H // _WCHUNK
        stgs = (stg0_ref, stg1_ref)
        sems = (sem0, sem1)

        def copy(k):
            return pltpu.make_async_copy(
                w1c_ref.at[pl.ds(k * _WCHUNK, _WCHUNK), :],
                stgs[k % 2], sems[k % 2])

        copy(0).start()
        for k in range(nk):
            if k + 1 < nk:
                copy(k + 1).start()
            copy(k).wait()
            w1s_ref[pl.ds(k * _WCHUNK, _WCHUNK), :] = (
                stgs[k % 2][...].astype(jnp.bfloat16))

    xb = x_ref[...].astype(jnp.bfloat16)
    logits = jax.lax.dot_general(
        xb, gw_ref[...], (((1,), (1,)), ((), ())),
        preferred_element_type=jnp.float32,
    ) + gb_ref[...]
    logits_ref[...] = logits

    cols = jax.lax.broadcasted_iota(jnp.int32, (BN, E), 1)
    m1 = jnp.max(logits, axis=1, keepdims=True)
    i1 = jnp.min(jnp.where(logits == m1, cols, E), axis=1, keepdims=True)
    masked = jnp.where(cols == i1, -jnp.inf, logits)
    m2 = jnp.max(masked, axis=1, keepdims=True)
    i2 = jnp.min(jnp.where(masked == m2, cols, E), axis=1, keepdims=True)
    s = jax.nn.sigmoid(m1 - m2)

    a = jax.lax.dot_general(
        xb, w1s_ref[...], (((1,), (1,)), ((), ())),
        preferred_element_type=jnp.float32,
    ) + b1c_ref[...]
    g = (0.5 * a * (1.0 + jax.lax.erf(a * _INV_SQRT2))
         ).astype(jnp.bfloat16)
    pscores = jax.lax.dot_general(
        g, w2b_ref[...], (((1,), (0,)), ((), ())),
        preferred_element_type=jnp.float32,
    )
    wexp = (jnp.where(cols == i1, s, 0.0)
            + jnp.where(cols == i2, 1.0 - s, 0.0))
    scores_ref[...] = jnp.sum(pscores * wexp, axis=1, keepdims=True)


@functools.partial(jax.jit, static_argnames=())
def kernel(x, gate_w, gate_b, w1, b1, w2):
    w1c = w1.reshape(E * H, D)
    b1c = b1.reshape(1, E * H)
    idx = jnp.arange(E)
    w2b = (jnp.zeros((E, H, E), jnp.float32)
           .at[idx, :, idx].set(w2.reshape(E, H))
           .reshape(E * H, E).astype(jnp.bfloat16))
    gwb = gate_w.astype(jnp.bfloat16)
    gb2 = gate_b.reshape(1, E)
    grid = (N_TOKENS // BN,)
    scores, logits = pl.pallas_call(
        _moe_head_kernel,
        grid=grid,
        in_specs=[
            pl.BlockSpec((BN, D), lambda i: (i, 0)),
            pl.BlockSpec((E, D), lambda i: (0, 0)),
            pl.BlockSpec((1, E), lambda i: (0, 0)),
            pl.BlockSpec(memory_space=pl.ANY),
            pl.BlockSpec((1, E * H), lambda i: (0, 0)),
            pl.BlockSpec((E * H, E), lambda i: (0, 0)),
        ],
        out_specs=[
            pl.BlockSpec((BN, 1), lambda i: (i, 0)),
            pl.BlockSpec((BN, E), lambda i: (i, 0)),
        ],
        out_shape=[
            jax.ShapeDtypeStruct((N_TOKENS, 1), jnp.float32),
            jax.ShapeDtypeStruct((N_TOKENS, E), jnp.float32),
        ],
        scratch_shapes=[
            pltpu.VMEM((E * H, D), jnp.bfloat16),
            pltpu.VMEM((_WCHUNK, D), jnp.float32),
            pltpu.VMEM((_WCHUNK, D), jnp.float32),
            pltpu.SemaphoreType.DMA,
            pltpu.SemaphoreType.DMA,
        ],
        compiler_params=pltpu.CompilerParams(
            dimension_semantics=("arbitrary",),
            vmem_limit_bytes=120 * 1024 * 1024,
        ),
    )(x, gwb, gb2, w1c, b1c, w2b)
    return scores, logits
